# Initial kernel scaffold; baseline (speedup 1.0000x reference)
import jax
import jax.numpy as jnp
from jax.experimental import pallas as pl


def kernel(x, time_indices, pe, emb_1, emb_5, emb_15, emb_60, temporal_importance):
    raise NotImplementedError("write your pallas kernel here")


# speedup vs baseline: 13.7917x; 13.7917x over previous
import functools

import jax
import jax.numpy as jnp
from jax.experimental import pallas as pl
from jax.experimental.pallas import tpu as pltpu
from jax.experimental.pallas import tpu_sc as plsc

_SCALES = (1, 5, 15, 60)

_GATHER_WINDOW = 128
_TC_BLOCK_B = 32


def _build_combined(max_seq, embs, w):
    parts = []
    for i, (emb, scale) in enumerate(zip(embs, _SCALES)):
        rep = jnp.repeat(emb, scale, axis=0)[:max_seq]
        parts.append(rep * w[i])
    return jnp.concatenate(parts, axis=1)


def _sc_gather(table, idx_flat, n_rows, d):
    mesh = plsc.VectorSubcoreMesh(core_axis_name="c", subcore_axis_name="s")

    @functools.partial(
        pl.kernel,
        out_type=jax.ShapeDtypeStruct((n_rows, d), jnp.float32),
        mesh=mesh,
    )
    def gather_kernel(tbl_hbm, idx_hbm, out_hbm):
        def body(idx_vmem, out_vmem):
            pltpu.sync_copy(tbl_hbm.at[idx_vmem.at[0]], out_vmem)

        pltpu.emit_pipeline(
            body,
            grid=(n_rows // _GATHER_WINDOW,),
            in_specs=[pl.BlockSpec((1, _GATHER_WINDOW), lambda i: (0, i))],
            out_specs=[pl.BlockSpec((_GATHER_WINDOW, d), lambda i: (i, 0))],
            core_axis_name=("c", "s"),
            dimension_semantics=(pltpu.PARALLEL,),
        )(idx_hbm, out_hbm)

    return gather_kernel(table, idx_flat)


def _tc_add(x, g, pe_s):
    b, s, d = x.shape

    def body(x_ref, g_ref, pe_ref, o_ref):
        o_ref[...] = x_ref[...] + (pe_ref[...] + g_ref[...])

    return pl.pallas_call(
        body,
        out_shape=jax.ShapeDtypeStruct((b, s, d), x.dtype),
        grid=(b // _TC_BLOCK_B,),
        in_specs=[
            pl.BlockSpec((_TC_BLOCK_B, s, d), lambda i: (i, 0, 0)),
            pl.BlockSpec((_TC_BLOCK_B, s, d), lambda i: (i, 0, 0)),
            pl.BlockSpec((s, d), lambda i: (0, 0)),
        ],
        out_specs=pl.BlockSpec((_TC_BLOCK_B, s, d), lambda i: (i, 0, 0)),
    )(x, g, pe_s)


def kernel(x, time_indices, pe, emb_1, emb_5, emb_15, emb_60, temporal_importance):
    b, s, d = x.shape
    max_seq = pe.shape[0]
    combined = _build_combined(
        max_seq, (emb_1, emb_5, emb_15, emb_60), temporal_importance
    )
    idx = time_indices.reshape(1, b * s).astype(jnp.int32)
    g = _sc_gather(combined, idx, b * s, d).reshape(b, s, d)
    return _tc_add(x, g, pe[:s])
